# baseline (speedup 1.0000x reference)
import jax
import jax.numpy as jnp
from jax.experimental import pallas as pl
from jax.experimental.pallas import tpu as pltpu

_N_STREAMS = 1


def _gru_chunk_kernel(x_ref, w_zr_ref, b_zr_ref, wh_n_ref, wx_n_ref, b_n_ref,
                      hist_ref, h_carry):
    batch, t_tile, _ = x_ref.shape
    x_t = x_ref[...].astype(jnp.bfloat16).transpose(1, 0, 2)
    hidden = wh_n_ref.shape[1]

    @pl.when(pl.program_id(0) == 0)
    def _():
        h_carry[...] = jnp.zeros_like(h_carry)

    w_zr = w_zr_ref[...]
    b_zr = b_zr_ref[...]
    wh_n = wh_n_ref[...]
    wx_n = wx_n_ref[...]
    b_n = b_n_ref[...]

    bs = batch // 2
    hs = [h_carry[s * bs:(s + 1) * bs, :] for s in range(2)]
    for t in range(t_tile):
        xt = [x_t[t, s * bs:(s + 1) * bs, :] for s in range(2)]
        tzr = [jnp.tanh(
                   jnp.dot(jnp.concatenate(
                               [hs[s].astype(jnp.bfloat16), xt[s]], axis=1),
                           w_zr, preferred_element_type=jnp.float32) + b_zr)
               for s in range(2)]
        rh = [(0.5 + 0.5 * tzr[s][:, hidden:]) * hs[s] for s in range(2)]
        an = [jnp.dot(rh[s].astype(jnp.bfloat16), wh_n,
                      preferred_element_type=jnp.float32)
              + jnp.dot(xt[s], wx_n, preferred_element_type=jnp.float32)
              + b_n
              for s in range(2)]
        for s in range(2):
            n = jnp.tanh(an[s])
            z = 0.5 + 0.5 * tzr[s][:, :hidden]
            hs[s] = hs[s] + z * (n - hs[s])
            hist_ref[t, s * bs:(s + 1) * bs, :] = hs[s]

    for s in range(2):
        h_carry[s * bs:(s + 1) * bs, :] = hs[s]


def _largest_divisor_leq(n, cap):
    for cand in range(min(n, cap), 0, -1):
        if n % cand == 0:
            return cand
    return 1


@jax.jit
def kernel(x_btd, wz, bz, wr, br, wn, bn):
    B, T, D = x_btd.shape
    H = wz.shape[1]
    if T == 0:
        return jnp.zeros((0, B, H), jnp.float32)

    w_zr = (0.5 * jnp.concatenate([wz, wr], axis=1)).astype(jnp.bfloat16)
    b_zr = 0.5 * jnp.concatenate([bz, br], axis=1)
    wh_n = wn[:H].astype(jnp.bfloat16)
    wx_n = wn[H:].astype(jnp.bfloat16)

    t_tile = _largest_divisor_leq(T, 32)
    grid = (T // t_tile,)

    return pl.pallas_call(
        _gru_chunk_kernel,
        out_shape=jax.ShapeDtypeStruct((T, B, H), jnp.float32),
        grid=grid,
        in_specs=[
            pl.BlockSpec((B, t_tile, D), lambda i: (0, i, 0)),
            pl.BlockSpec((H + D, 2 * H), lambda i: (0, 0)),
            pl.BlockSpec((1, 2 * H), lambda i: (0, 0)),
            pl.BlockSpec((H, H), lambda i: (0, 0)),
            pl.BlockSpec((D, H), lambda i: (0, 0)),
            pl.BlockSpec((1, H), lambda i: (0, 0)),
        ],
        out_specs=pl.BlockSpec((t_tile, B, H), lambda i: (i, 0, 0)),
        scratch_shapes=[pltpu.VMEM((B, H), jnp.float32)],
        compiler_params=pltpu.CompilerParams(
            dimension_semantics=("arbitrary",)),
    )(x_btd, w_zr, b_zr, wh_n, wx_n, bn)


# speedup vs baseline: 1.0297x; 1.0297x over previous
import jax
import jax.numpy as jnp
from jax.experimental import pallas as pl
from jax.experimental.pallas import tpu as pltpu

_N_STREAMS = 1


def _gru_chunk_kernel(x_ref, w_zr_ref, b_zr_ref, wh_n_ref, wx_n_ref, b_n_ref,
                      hist_ref, h_carry):
    batch, t_tile, _ = x_ref.shape
    x_t = x_ref[...].astype(jnp.bfloat16).transpose(1, 0, 2)
    hidden = wh_n_ref.shape[1]

    @pl.when(pl.program_id(0) == 0)
    def _():
        h_carry[...] = jnp.zeros_like(h_carry)

    w_zr = w_zr_ref[...]
    b_zr = b_zr_ref[...]
    wh_n = wh_n_ref[...]
    wx_n = wx_n_ref[...]
    b_n = b_n_ref[...]

    bs = batch // 2
    hs = [h_carry[s * bs:(s + 1) * bs, :] for s in range(2)]
    for t in range(t_tile):
        xt = [x_t[t, s * bs:(s + 1) * bs, :] for s in range(2)]
        tzr = [jnp.tanh(
                   jnp.dot(jnp.concatenate(
                               [hs[s].astype(jnp.bfloat16), xt[s]], axis=1),
                           w_zr, preferred_element_type=jnp.float32) + b_zr)
               for s in range(2)]
        rh = [(0.5 + 0.5 * tzr[s][:, hidden:]) * hs[s] for s in range(2)]
        an = [jnp.dot(rh[s].astype(jnp.bfloat16), wh_n,
                      preferred_element_type=jnp.float32)
              + jnp.dot(xt[s], wx_n, preferred_element_type=jnp.float32)
              + b_n
              for s in range(2)]
        for s in range(2):
            n = jnp.tanh(an[s])
            z = 0.5 + 0.5 * tzr[s][:, :hidden]
            hs[s] = hs[s] + z * (n - hs[s])
            hist_ref[t, s * bs:(s + 1) * bs, :] = hs[s]

    for s in range(2):
        h_carry[s * bs:(s + 1) * bs, :] = hs[s]


def _largest_divisor_leq(n, cap):
    for cand in range(min(n, cap), 0, -1):
        if n % cand == 0:
            return cand
    return 1


@jax.jit
def kernel(x_btd, wz, bz, wr, br, wn, bn):
    B, T, D = x_btd.shape
    H = wz.shape[1]
    if T == 0:
        return jnp.zeros((0, B, H), jnp.float32)

    w_zr = (0.5 * jnp.concatenate([wz, wr], axis=1)).astype(jnp.bfloat16)
    b_zr = 0.5 * jnp.concatenate([bz, br], axis=1)
    wh_n = wn[:H].astype(jnp.bfloat16)
    wx_n = wn[H:].astype(jnp.bfloat16)

    t_tile = _largest_divisor_leq(T, 8)
    grid = (T // t_tile,)

    return pl.pallas_call(
        _gru_chunk_kernel,
        out_shape=jax.ShapeDtypeStruct((T, B, H), jnp.float32),
        grid=grid,
        in_specs=[
            pl.BlockSpec((B, t_tile, D), lambda i: (0, i, 0)),
            pl.BlockSpec((H + D, 2 * H), lambda i: (0, 0)),
            pl.BlockSpec((1, 2 * H), lambda i: (0, 0)),
            pl.BlockSpec((H, H), lambda i: (0, 0)),
            pl.BlockSpec((D, H), lambda i: (0, 0)),
            pl.BlockSpec((1, H), lambda i: (0, 0)),
        ],
        out_specs=pl.BlockSpec((t_tile, B, H), lambda i: (i, 0, 0)),
        scratch_shapes=[pltpu.VMEM((B, H), jnp.float32)],
        compiler_params=pltpu.CompilerParams(
            dimension_semantics=("arbitrary",)),
    )(x_btd, w_zr, b_zr, wh_n, wx_n, bn)


# speedup vs baseline: 1.0317x; 1.0020x over previous
import jax
import jax.numpy as jnp
from jax.experimental import pallas as pl
from jax.experimental.pallas import tpu as pltpu


def _gru_chunk_kernel(x_ref, w_zr_ref, b_zr_ref, wh_n_ref, wx_n_ref, b_n_ref,
                      hist_ref, h_carry):
    batch, t_tile, _ = x_ref.shape
    x_t = x_ref[...].astype(jnp.bfloat16).transpose(1, 0, 2)
    hidden = wh_n_ref.shape[1]

    @pl.when(pl.program_id(0) == 0)
    def _():
        h_carry[...] = jnp.zeros_like(h_carry)

    w_zr = w_zr_ref[...]
    b_zr = b_zr_ref[...]
    wh_n = wh_n_ref[...]
    wx_n = wx_n_ref[...]
    b_n = b_n_ref[...]

    bs = batch // 2
    rows = [(0, bs), (bs, batch)]
    hs = [h_carry[lo:hi, :] for lo, hi in rows]
    for t in range(t_tile):
        xt = [x_t[t, lo:hi, :] for lo, hi in rows]
        tzr = [jnp.tanh(
                   jnp.dot(jnp.concatenate(
                               [hs[s].astype(jnp.bfloat16), xt[s]], axis=1),
                           w_zr, preferred_element_type=jnp.float32) + b_zr)
               for s in range(2)]
        rh = [(0.5 + 0.5 * tzr[s][:, hidden:]) * hs[s] for s in range(2)]
        an = [jnp.dot(rh[s].astype(jnp.bfloat16), wh_n,
                      preferred_element_type=jnp.float32)
              + jnp.dot(xt[s], wx_n, preferred_element_type=jnp.float32)
              + b_n
              for s in range(2)]
        for s, (lo, hi) in enumerate(rows):
            n = jnp.tanh(an[s])
            z = 0.5 + 0.5 * tzr[s][:, :hidden]
            hs[s] = hs[s] + z * (n - hs[s])
            hist_ref[t, lo:hi, :] = hs[s]

    for s, (lo, hi) in enumerate(rows):
        h_carry[lo:hi, :] = hs[s]


def _largest_divisor_leq(n, cap):
    for cand in range(min(n, cap), 0, -1):
        if n % cand == 0:
            return cand
    return 1


@jax.jit
def kernel(x_btd, wz, bz, wr, br, wn, bn):
    B, T, D = x_btd.shape
    H = wz.shape[1]
    if T == 0:
        return jnp.zeros((0, B, H), jnp.float32)

    w_zr = (0.5 * jnp.concatenate([wz, wr], axis=1)).astype(jnp.bfloat16)
    b_zr = 0.5 * jnp.concatenate([bz, br], axis=1)
    wh_n = wn[:H].astype(jnp.bfloat16)
    wx_n = wn[H:].astype(jnp.bfloat16)

    t_tile = _largest_divisor_leq(T, 8)
    grid = (T // t_tile,)

    return pl.pallas_call(
        _gru_chunk_kernel,
        out_shape=jax.ShapeDtypeStruct((T, B, H), jnp.float32),
        grid=grid,
        in_specs=[
            pl.BlockSpec((B, t_tile, D), lambda i: (0, i, 0)),
            pl.BlockSpec((H + D, 2 * H), lambda i: (0, 0)),
            pl.BlockSpec((1, 2 * H), lambda i: (0, 0)),
            pl.BlockSpec((H, H), lambda i: (0, 0)),
            pl.BlockSpec((D, H), lambda i: (0, 0)),
            pl.BlockSpec((1, H), lambda i: (0, 0)),
        ],
        out_specs=pl.BlockSpec((t_tile, B, H), lambda i: (i, 0, 0)),
        scratch_shapes=[pltpu.VMEM((B, H), jnp.float32)],
        compiler_params=pltpu.CompilerParams(
            dimension_semantics=("arbitrary",)),
    )(x_btd, w_zr, b_zr, wh_n, wx_n, bn)


# speedup vs baseline: 1.1932x; 1.1566x over previous
import jax
import jax.numpy as jnp
from jax.experimental import pallas as pl
from jax.experimental.pallas import tpu as pltpu


def _gru_chunk_kernel(x_ref, w_zr_ref, b_zr_ref, wh_n_ref, wx_n_ref, b_n_ref,
                      hist_ref, h_carry):
    batch, t_tile, _ = x_ref.shape
    x_t = x_ref[...].astype(jnp.bfloat16).transpose(1, 0, 2)
    hidden = wh_n_ref.shape[1]

    @pl.when(pl.program_id(0) == 0)
    def _():
        h_carry[...] = jnp.zeros_like(h_carry)

    w_zr = w_zr_ref[...]
    b_zr = b_zr_ref[...]
    wh_n = wh_n_ref[...]
    wx_n = wx_n_ref[...]
    b_n = b_n_ref[...]

    bs = batch // 2
    rows = [(0, bs), (bs, batch)]
    hs = [h_carry[lo:hi, :] for lo, hi in rows]
    for t in range(t_tile):
        xt = [x_t[t, lo:hi, :] for lo, hi in rows]
        tzr = [jnp.tanh(
                   jnp.dot(hs[s].astype(jnp.bfloat16), w_zr[:hidden, :],
                           preferred_element_type=jnp.float32)
                   + jnp.dot(xt[s], w_zr[hidden:, :],
                             preferred_element_type=jnp.float32)
                   + b_zr)
               for s in range(2)]
        rh = [(0.5 + 0.5 * tzr[s][:, hidden:]) * hs[s] for s in range(2)]
        an = [jnp.dot(rh[s].astype(jnp.bfloat16), wh_n,
                      preferred_element_type=jnp.float32)
              + jnp.dot(xt[s], wx_n, preferred_element_type=jnp.float32)
              + b_n
              for s in range(2)]
        for s, (lo, hi) in enumerate(rows):
            n = jnp.tanh(an[s])
            z = 0.5 + 0.5 * tzr[s][:, :hidden]
            hs[s] = hs[s] + z * (n - hs[s])
            hist_ref[t, lo:hi, :] = hs[s]

    for s, (lo, hi) in enumerate(rows):
        h_carry[lo:hi, :] = hs[s]


def _largest_divisor_leq(n, cap):
    for cand in range(min(n, cap), 0, -1):
        if n % cand == 0:
            return cand
    return 1


@jax.jit
def kernel(x_btd, wz, bz, wr, br, wn, bn):
    B, T, D = x_btd.shape
    H = wz.shape[1]
    if T == 0:
        return jnp.zeros((0, B, H), jnp.float32)

    w_zr = (0.5 * jnp.concatenate([wz, wr], axis=1)).astype(jnp.bfloat16)
    b_zr = 0.5 * jnp.concatenate([bz, br], axis=1)
    wh_n = wn[:H].astype(jnp.bfloat16)
    wx_n = wn[H:].astype(jnp.bfloat16)

    t_tile = _largest_divisor_leq(T, 8)
    grid = (T // t_tile,)

    return pl.pallas_call(
        _gru_chunk_kernel,
        out_shape=jax.ShapeDtypeStruct((T, B, H), jnp.float32),
        grid=grid,
        in_specs=[
            pl.BlockSpec((B, t_tile, D), lambda i: (0, i, 0)),
            pl.BlockSpec((H + D, 2 * H), lambda i: (0, 0)),
            pl.BlockSpec((1, 2 * H), lambda i: (0, 0)),
            pl.BlockSpec((H, H), lambda i: (0, 0)),
            pl.BlockSpec((D, H), lambda i: (0, 0)),
            pl.BlockSpec((1, H), lambda i: (0, 0)),
        ],
        out_specs=pl.BlockSpec((t_tile, B, H), lambda i: (i, 0, 0)),
        scratch_shapes=[pltpu.VMEM((B, H), jnp.float32)],
        compiler_params=pltpu.CompilerParams(
            dimension_semantics=("arbitrary",)),
    )(x_btd, w_zr, b_zr, wh_n, wx_n, bn)
